# baseline (speedup 1.0000x reference)
import functools

import jax
import jax.numpy as jnp
from jax import lax
from jax.experimental import pallas as pl
from jax.experimental.pallas import tpu as pltpu
from jax.experimental.pallas import tpu_sc as plsc


@functools.lru_cache(maxsize=None)
def _build(B, S, H, V):
    info = plsc.get_sparse_core_info()
    NC, NS = info.num_cores, info.num_subcores
    NW = NC * NS
    L = 16
    BT = 128
    assert B // BT == NW
    HT = H // 8
    assert H % L == 0 and S % 2 == 0
    UNROLL = 8
    assert BT % UNROLL == 0

    mesh = plsc.VectorSubcoreMesh(core_axis_name="c", subcore_axis_name="s")

    @functools.partial(
        pl.kernel,
        out_type=jax.ShapeDtypeStruct((S, HT, NW, 8 * BT), jnp.float32),
        mesh=mesh,
        compiler_params=pltpu.CompilerParams(
            use_tc_tiling_on_sc=False, needs_layout_passes=False),
        scratch_types=[
            pltpu.VMEM((S, BT), jnp.int32),
            pltpu.VMEM((BT, H), jnp.float32),
            pltpu.VMEM((BT, H), jnp.float32),
            pltpu.VMEM((HT * 8 * BT,), jnp.float32),
            pltpu.VMEM((HT * 8 * BT,), jnp.float32),
            pltpu.VMEM((S, H), jnp.float32),
            pltpu.SemaphoreType.DMA,
            pltpu.SemaphoreType.DMA,
            pltpu.SemaphoreType.DMA,
            pltpu.SemaphoreType.DMA,
        ],
    )
    def k(xt_hbm, emb_hbm, pos_hbm, out_hbm,
          idxall, rows0, rows1, tr0, tr1, pos_v, sg0, sg1, so0, so1):
        wid = lax.axis_index("s") * NC + lax.axis_index("c")
        pltpu.sync_copy(xt_hbm.at[:, pl.ds(wid * BT, BT)], idxall)
        pltpu.sync_copy(pos_hbm, pos_v)
        rows = (rows0, rows1)
        tr = (tr0, tr1)
        sg = (sg0, sg1)
        so = (so0, so1)
        iota = lax.iota(jnp.int32, L)
        m15 = jnp.full((L,), L - 1, jnp.int32)
        m7 = jnp.full((L,), 7, jnp.int32)
        perm = [jax.lax.bitwise_and(iota + d, m15) for d in range(L)]

        def gather_start(s, b):
            pltpu.make_async_copy(
                emb_hbm.at[idxall.at[s]], rows[b], sg[b]).start()

        def gather_wait(b):
            pltpu.make_async_copy(
                emb_hbm.at[idxall.at[0]], rows[b], sg[b]).wait()

        TILE = 8 * BT

        def out_start(s, b):
            for ht in range(HT):
                pltpu.make_async_copy(
                    tr[b].at[pl.ds(ht * TILE, TILE)],
                    out_hbm.at[s, ht, wid], so[b]).start()

        def out_wait(b):
            for ht in range(HT):
                pltpu.make_async_copy(
                    tr[b].at[pl.ds(ht * TILE, TILE)],
                    out_hbm.at[0, ht, wid], so[b]).wait()

        dnums = jax.lax.GatherDimensionNumbers(
            offset_dims=(), collapsed_slice_dims=(0,),
            start_index_map=(0,))

        def take16(vec, idx):
            return jax.lax.gather(
                vec, idx[:, None], dnums, (1,),
                mode=jax.lax.GatherScatterMode.PROMISE_IN_BOUNDS)

        def compute(s, b):
            rv_ref, tr_ref = rows[b], tr[b]
            pv = [pos_v[s, pl.ds(j * L, L)] for j in range(H // L)]

            def blk_body(i, carry):
                t0 = i * L
                rowv = iota + t0
                for j0 in range(0, H // L, 4):
                    jds = [(j0 + jj, d) for jj in range(4) for d in range(L)]
                    hvs = {jd: perm[jd[1]] + jd[0] * L for jd in jds}
                    dvecs = {jd: plsc.load_gather(rv_ref, [rowv, hvs[jd]])
                             for jd in jds}
                    for jd in jds:
                        j, d = jd
                        dvec = dvecs[jd] + take16(pv[j], perm[d])
                        plsc.store_scatter(
                            tr_ref,
                            [jax.lax.shift_left(
                                hvs[jd],
                                jnp.full((L,), 7, jnp.int32)) + rowv],
                            dvec)
                return carry

            lax.fori_loop(0, BT // L, blk_body, 0)

        gather_start(0, 0)

        def step(i, carry):
            for b in range(2):
                s = 2 * i + b
                nxt = s + 1

                @pl.when(nxt < S)
                def _():
                    gather_start(nxt, 1 - b)

                gather_wait(b)

                @pl.when(s >= 2)
                def _():
                    out_wait(b)

                compute(s, b)
                out_start(s, b)
            return carry

        lax.fori_loop(0, S // 2, step, 0)
        out_wait(0)
        out_wait(1)

    return k


def kernel(x, emb_table, pos_table):
    B, S = x.shape
    V, H = emb_table.shape
    k = _build(B, S, H, V)
    xt = jnp.swapaxes(x.astype(jnp.int32), 0, 1)
    out4 = k(xt, emb_table, pos_table)
    out5 = out4.reshape(S, H // 8, B // 128, 8, 128)
    return jnp.transpose(out5, (2, 4, 0, 1, 3)).reshape(B, S, H)


# speedup vs baseline: 1.2668x; 1.2668x over previous
import functools

import jax
import jax.numpy as jnp
from jax import lax
from jax.experimental import pallas as pl
from jax.experimental.pallas import tpu as pltpu
from jax.experimental.pallas import tpu_sc as plsc


@functools.lru_cache(maxsize=None)
def _build(B, S, H, V):
    info = plsc.get_sparse_core_info()
    NC, NS = info.num_cores, info.num_subcores
    NW = NC * NS
    L = 16
    BT = 128
    assert B // BT == NW
    HT = H // 8
    assert H % L == 0 and S % 2 == 0
    UNROLL = 8
    assert BT % UNROLL == 0

    mesh = plsc.VectorSubcoreMesh(core_axis_name="c", subcore_axis_name="s")

    @functools.partial(
        pl.kernel,
        out_type=jax.ShapeDtypeStruct((S, HT, NW, 8 * BT), jnp.float32),
        mesh=mesh,
        compiler_params=pltpu.CompilerParams(
            use_tc_tiling_on_sc=False, needs_layout_passes=False),
        scratch_types=[
            pltpu.VMEM((S, BT), jnp.int32),
            pltpu.VMEM((BT, H), jnp.float32),
            pltpu.VMEM((BT, H), jnp.float32),
            pltpu.VMEM((HT * 8 * BT,), jnp.float32),
            pltpu.VMEM((HT * 8 * BT,), jnp.float32),
            pltpu.VMEM((S, H), jnp.float32),
            pltpu.SemaphoreType.DMA,
            pltpu.SemaphoreType.DMA,
            pltpu.SemaphoreType.DMA,
            pltpu.SemaphoreType.DMA,
        ],
    )
    def k(xt_hbm, emb_hbm, pos_hbm, out_hbm,
          idxall, rows0, rows1, tr0, tr1, pos_v, sg0, sg1, so0, so1):
        wid = lax.axis_index("s") * NC + lax.axis_index("c")
        pltpu.sync_copy(xt_hbm.at[:, pl.ds(wid * BT, BT)], idxall)
        pltpu.sync_copy(pos_hbm, pos_v)
        rows = (rows0, rows1)
        tr = (tr0, tr1)
        sg = (sg0, sg1)
        so = (so0, so1)
        iota = lax.iota(jnp.int32, L)
        m15 = jnp.full((L,), L - 1, jnp.int32)
        perm = [jax.lax.bitwise_and(iota + d, m15) for d in range(L)]
        io64p = [iota * H + perm[d] for d in range(L)]
        pio = [jax.lax.shift_left(perm[d], jnp.full((L,), 7, jnp.int32))
               + iota for d in range(L)]

        def gather_start(s, b):
            pltpu.make_async_copy(
                emb_hbm.at[idxall.at[s]], rows[b], sg[b]).start()

        def gather_wait(b):
            pltpu.make_async_copy(
                emb_hbm.at[idxall.at[0]], rows[b], sg[b]).wait()

        TILE = 8 * BT

        def out_start(s, b):
            for ht in range(HT):
                pltpu.make_async_copy(
                    tr[b].at[pl.ds(ht * TILE, TILE)],
                    out_hbm.at[s, ht, wid], so[b]).start()

        def out_wait(b):
            for ht in range(HT):
                pltpu.make_async_copy(
                    tr[b].at[pl.ds(ht * TILE, TILE)],
                    out_hbm.at[0, ht, wid], so[b]).wait()

        dnums = jax.lax.GatherDimensionNumbers(
            offset_dims=(), collapsed_slice_dims=(0,),
            start_index_map=(0,))

        def take16(vec, idx):
            return jax.lax.gather(
                vec, idx[:, None], dnums, (1,),
                mode=jax.lax.GatherScatterMode.PROMISE_IN_BOUNDS)

        def compute(s, b):
            rv_ref, tr_ref = rows[b], tr[b]
            pv = [pos_v[s, pl.ds(j * L, L)] for j in range(H // L)]

            def blk_body(i, carry):
                t0 = i * L
                for j in range(H // L):
                    rowv = iota + t0
                    doff = j * L * BT + t0
                    dvecs = [plsc.load_gather(rv_ref,
                                              [rowv, perm[d] + j * L])
                             for d in range(L)]
                    for d in range(L):
                        plsc.store_scatter(
                            tr_ref, [pio[d] + doff],
                            dvecs[d] + take16(pv[j], perm[d]))
                return carry

            lax.fori_loop(0, BT // L, blk_body, 0)

        gather_start(0, 0)

        def step(i, carry):
            for b in range(2):
                s = 2 * i + b
                nxt = s + 1

                @pl.when(nxt < S)
                def _():
                    gather_start(nxt, 1 - b)

                gather_wait(b)

                @pl.when(s >= 2)
                def _():
                    out_wait(b)

                compute(s, b)
                out_start(s, b)
            return carry

        lax.fori_loop(0, S // 2, step, 0)
        out_wait(0)
        out_wait(1)

    return k


def kernel(x, emb_table, pos_table):
    B, S = x.shape
    V, H = emb_table.shape
    k = _build(B, S, H, V)
    xt = jnp.swapaxes(x.astype(jnp.int32), 0, 1)
    out4 = k(xt, emb_table, pos_table)
    out5 = out4.reshape(S, H // 8, B // 128, 8, 128)
    return jnp.transpose(out5, (2, 4, 0, 1, 3)).reshape(B, S, H)
